# Initial kernel scaffold; baseline (speedup 1.0000x reference)
import jax
import jax.numpy as jnp
from jax.experimental import pallas as pl


def kernel(node_input, node_attr, edge_src, edge_dst, edge_attr, edge_length_embedded, W_si, W_lin1, W_fc1, W_fc2, W_lin2):
    raise NotImplementedError("write your pallas kernel here")


# speedup vs baseline: 2.1880x; 2.1880x over previous
import functools

import numpy as np
import jax
import jax.numpy as jnp
from jax import lax
from jax.experimental import pallas as pl
from jax.experimental.pallas import tpu as pltpu
from jax.experimental.pallas import tpu_sc as plsc

_N = 10000
_E = 320000
_D = 128
_NB = 8
_RN = 64

_NC = 2
_NS = 16
_L = 16
_NTILES = _NC * _NS
_EPT = _E // _NTILES
_K = 80
_NCHUNK = _EPT // _K
_RPS = 632

_IS128 = float(1.0 / np.sqrt(128.0))
_IS64 = float(1.0 / np.sqrt(64.0))
_IS8 = float(1.0 / np.sqrt(8.0))
_ALPHA = float(0.5 / (np.sqrt(16.0) * np.sqrt(128.0)))

_NODE_BLK = 2000
_EDGE_BLK = 8000


def _node_pre_body(ni, na, wsi, wlin1, x_o, si_o):
    xa = ni[...] * na[...]
    x_o[...] = jnp.dot(xa, wlin1[...], preferred_element_type=jnp.float32) * _IS128
    si_o[...] = jnp.dot(xa, wsi[...], preferred_element_type=jnp.float32) * _IS128


def _edge_pre_body(elb, ea, w1, w2, wp_o):
    h = jnp.dot(elb[...], w1[...], preferred_element_type=jnp.float32) * _IS8
    h = jnp.maximum(h, 0.0)
    w = jnp.dot(h, w2[...], preferred_element_type=jnp.float32) * _IS64
    wp_o[...] = w * ea[...]


def _post_body(p, si, wl2, o):
    agg = p[0] + p[1]
    o[...] = si[...] + _ALPHA * jnp.dot(agg, wl2[...], preferred_element_type=jnp.float32)


def _sc_scatter_body(wp_hbm, x_hbm, src_hbm, dst_hbm, zeros_hbm, out_hbm,
                     sidx_v, didx_v, w_v, xr_v, agg_sh, sem):
    c = lax.axis_index("c")
    s = lax.axis_index("s")
    row0 = jnp.minimum(s * _RPS, _N - _RPS)
    pltpu.sync_copy(zeros_hbm.at[pl.ds(row0, _RPS)],
                    agg_sh.at[pl.ds(row0, _RPS)])
    plsc.subcore_barrier()
    wid = c * _NS + s
    base0 = wid * _EPT

    @pl.loop(0, _NCHUNK)
    def _chunk(ci):
        base = base0 + ci * _K
        pltpu.sync_copy(src_hbm.at[pl.ds(base, _K)], sidx_v)
        pltpu.sync_copy(dst_hbm.at[pl.ds(base, _K)], didx_v)
        pltpu.sync_copy(wp_hbm.at[pl.ds(base, _K)], w_v)
        pltpu.async_copy(x_hbm.at[sidx_v], xr_v, sem).wait()

        @pl.loop(0, _K)
        def _row(r):
            for j in range(_D // _L):
                sl = pl.ds(j * _L, _L)
                w_v.at[r, sl][...] = w_v.at[r, sl][...] * xr_v.at[r, sl][...]

        pltpu.sync_copy(w_v, agg_sh.at[didx_v], add=True)

    plsc.subcore_barrier()
    pltpu.sync_copy(agg_sh.at[pl.ds(row0, _RPS)],
                    out_hbm.at[c, pl.ds(row0, _RPS)])


def kernel(node_input, node_attr, edge_src, edge_dst, edge_attr,
           edge_length_embedded, W_si, W_lin1, W_fc1, W_fc2, W_lin2):
    esrc = edge_src.astype(jnp.int32)
    edst = edge_dst.astype(jnp.int32)

    x, si = pl.pallas_call(
        _node_pre_body,
        grid=(_N // _NODE_BLK,),
        in_specs=[
            pl.BlockSpec((_NODE_BLK, _D), lambda i: (i, 0)),
            pl.BlockSpec((_NODE_BLK, 1), lambda i: (i, 0)),
            pl.BlockSpec((_D, _D), lambda i: (0, 0)),
            pl.BlockSpec((_D, _D), lambda i: (0, 0)),
        ],
        out_specs=[
            pl.BlockSpec((_NODE_BLK, _D), lambda i: (i, 0)),
            pl.BlockSpec((_NODE_BLK, _D), lambda i: (i, 0)),
        ],
        out_shape=[
            jax.ShapeDtypeStruct((_N, _D), jnp.float32),
            jax.ShapeDtypeStruct((_N, _D), jnp.float32),
        ],
    )(node_input, node_attr, W_si, W_lin1)

    wp = pl.pallas_call(
        _edge_pre_body,
        grid=(_E // _EDGE_BLK,),
        in_specs=[
            pl.BlockSpec((_EDGE_BLK, _NB), lambda i: (i, 0)),
            pl.BlockSpec((_EDGE_BLK, 1), lambda i: (i, 0)),
            pl.BlockSpec((_NB, _RN), lambda i: (0, 0)),
            pl.BlockSpec((_RN, _D), lambda i: (0, 0)),
        ],
        out_specs=pl.BlockSpec((_EDGE_BLK, _D), lambda i: (i, 0)),
        out_shape=jax.ShapeDtypeStruct((_E, _D), jnp.float32),
    )(edge_length_embedded, edge_attr, W_fc1, W_fc2)

    zeros = jnp.zeros((_N, _D), jnp.float32)

    mesh = plsc.VectorSubcoreMesh(core_axis_name="c", subcore_axis_name="s")
    sc_scatter = pl.kernel(
        _sc_scatter_body,
        mesh=mesh,
        out_type=jax.ShapeDtypeStruct((_NC, _N, _D), jnp.float32),
        scratch_types=[
            pltpu.VMEM((_K,), jnp.int32),
            pltpu.VMEM((_K,), jnp.int32),
            pltpu.VMEM((_K, _D), jnp.float32),
            pltpu.VMEM((_K, _D), jnp.float32),
            pltpu.VMEM_SHARED((_N, _D), jnp.float32),
            pltpu.SemaphoreType.DMA,
        ],
    )
    partial = sc_scatter(wp, x, esrc, edst, zeros)

    out = pl.pallas_call(
        _post_body,
        grid=(_N // _NODE_BLK,),
        in_specs=[
            pl.BlockSpec((_NC, _NODE_BLK, _D), lambda i: (0, i, 0)),
            pl.BlockSpec((_NODE_BLK, _D), lambda i: (i, 0)),
            pl.BlockSpec((_D, _D), lambda i: (0, 0)),
        ],
        out_specs=pl.BlockSpec((_NODE_BLK, _D), lambda i: (i, 0)),
        out_shape=jax.ShapeDtypeStruct((_N, _D), jnp.float32),
    )(partial, si, W_lin2)
    return out


# speedup vs baseline: 3.0901x; 1.4123x over previous
import functools

import numpy as np
import jax
import jax.numpy as jnp
from jax import lax
from jax.experimental import pallas as pl
from jax.experimental.pallas import tpu as pltpu
from jax.experimental.pallas import tpu_sc as plsc

_N = 10000
_E = 320000
_D = 128
_NB = 8
_RN = 64

_NC = 2
_NS = 16
_L = 16
_NTILES = _NC * _NS
_EPT = _E // _NTILES
_K = 80
_NCHUNK = _EPT // _K
_RPS = 632

_IS128 = float(1.0 / np.sqrt(128.0))
_IS64 = float(1.0 / np.sqrt(64.0))
_IS8 = float(1.0 / np.sqrt(8.0))
_ALPHA = float(0.5 / (np.sqrt(16.0) * np.sqrt(128.0)))

_NODE_BLK = 2000
_EDGE_BLK = 8000


def _node_pre_body(ni, na, wsi, wlin1, x_o, si_o):
    xa = ni[...] * na[...]
    x_o[...] = jnp.dot(xa, wlin1[...], preferred_element_type=jnp.float32) * _IS128
    si_o[...] = jnp.dot(xa, wsi[...], preferred_element_type=jnp.float32) * _IS128


def _edge_pre_body(elb, ea, w1, w2, wp_o):
    h = jnp.dot(elb[...], w1[...], preferred_element_type=jnp.float32) * _IS8
    h = jnp.maximum(h, 0.0)
    w = jnp.dot(h, w2[...], preferred_element_type=jnp.float32) * _IS64
    wp_o[...] = w * ea[...]


def _post_body(p, si, wl2, o):
    agg = p[0] + p[1]
    o[...] = si[...] + _ALPHA * jnp.dot(agg, wl2[...], preferred_element_type=jnp.float32)


def _sc_scatter_body(wp_hbm, x_hbm, src_hbm, dst_hbm, zeros_hbm, out_hbm,
                     sidx0, didx0, sidx1, didx1, w0, xr0, w1, xr1, agg_sh,
                     gs0, ws0, gs1, ws1):
    c = lax.axis_index("c")
    s = lax.axis_index("s")
    wid = c * _NS + s
    base0 = wid * _EPT
    row0 = jnp.minimum(s * _RPS, _N - _RPS)
    pltpu.sync_copy(zeros_hbm.at[pl.ds(row0, _RPS)],
                    agg_sh.at[pl.ds(row0, _RPS)])
    plsc.subcore_barrier()

    def issue(ci, sidx_v, didx_v, w_v, xr_v, gsem, wsem):
        base = base0 + ci * _K
        pltpu.sync_copy(src_hbm.at[pl.ds(base, _K)], sidx_v)
        pltpu.sync_copy(dst_hbm.at[pl.ds(base, _K)], didx_v)
        pltpu.async_copy(x_hbm.at[sidx_v], xr_v, gsem)
        pltpu.async_copy(wp_hbm.at[pl.ds(base, _K)], w_v, wsem)

    def process(ci, sidx_v, didx_v, w_v, xr_v, gsem, wsem):
        base = base0 + ci * _K
        pltpu.make_async_copy(x_hbm.at[sidx_v], xr_v, gsem).wait()
        pltpu.make_async_copy(wp_hbm.at[pl.ds(base, _K)], w_v, wsem).wait()

        @pl.loop(0, _K)
        def _row(r):
            for j in range(_D // _L):
                sl = pl.ds(j * _L, _L)
                w_v.at[r, sl][...] = w_v.at[r, sl][...] * xr_v.at[r, sl][...]

        pltpu.sync_copy(w_v, agg_sh.at[didx_v], add=True)

    issue(0, sidx0, didx0, w0, xr0, gs0, ws0)

    @pl.loop(0, (_NCHUNK - 1) // 2)
    def _pair(it):
        ci = it * 2
        issue(ci + 1, sidx1, didx1, w1, xr1, gs1, ws1)
        process(ci, sidx0, didx0, w0, xr0, gs0, ws0)
        issue(ci + 2, sidx0, didx0, w0, xr0, gs0, ws0)
        process(ci + 1, sidx1, didx1, w1, xr1, gs1, ws1)

    process(_NCHUNK - 1, sidx0, didx0, w0, xr0, gs0, ws0)

    plsc.subcore_barrier()
    pltpu.sync_copy(agg_sh.at[pl.ds(row0, _RPS)],
                    out_hbm.at[c, pl.ds(row0, _RPS)])


def kernel(node_input, node_attr, edge_src, edge_dst, edge_attr,
           edge_length_embedded, W_si, W_lin1, W_fc1, W_fc2, W_lin2):
    esrc = edge_src.astype(jnp.int32)
    edst = edge_dst.astype(jnp.int32)

    x, si = pl.pallas_call(
        _node_pre_body,
        grid=(_N // _NODE_BLK,),
        in_specs=[
            pl.BlockSpec((_NODE_BLK, _D), lambda i: (i, 0)),
            pl.BlockSpec((_NODE_BLK, 1), lambda i: (i, 0)),
            pl.BlockSpec((_D, _D), lambda i: (0, 0)),
            pl.BlockSpec((_D, _D), lambda i: (0, 0)),
        ],
        out_specs=[
            pl.BlockSpec((_NODE_BLK, _D), lambda i: (i, 0)),
            pl.BlockSpec((_NODE_BLK, _D), lambda i: (i, 0)),
        ],
        out_shape=[
            jax.ShapeDtypeStruct((_N, _D), jnp.float32),
            jax.ShapeDtypeStruct((_N, _D), jnp.float32),
        ],
    )(node_input, node_attr, W_si, W_lin1)

    wp = pl.pallas_call(
        _edge_pre_body,
        grid=(_E // _EDGE_BLK,),
        in_specs=[
            pl.BlockSpec((_EDGE_BLK, _NB), lambda i: (i, 0)),
            pl.BlockSpec((_EDGE_BLK, 1), lambda i: (i, 0)),
            pl.BlockSpec((_NB, _RN), lambda i: (0, 0)),
            pl.BlockSpec((_RN, _D), lambda i: (0, 0)),
        ],
        out_specs=pl.BlockSpec((_EDGE_BLK, _D), lambda i: (i, 0)),
        out_shape=jax.ShapeDtypeStruct((_E, _D), jnp.float32),
    )(edge_length_embedded, edge_attr, W_fc1, W_fc2)

    zeros = jnp.zeros((_N, _D), jnp.float32)

    mesh = plsc.VectorSubcoreMesh(core_axis_name="c", subcore_axis_name="s")
    sc_scatter = pl.kernel(
        _sc_scatter_body,
        mesh=mesh,
        out_type=jax.ShapeDtypeStruct((_NC, _N, _D), jnp.float32),
        scratch_types=[
            pltpu.VMEM((_K,), jnp.int32),
            pltpu.VMEM((_K,), jnp.int32),
            pltpu.VMEM((_K,), jnp.int32),
            pltpu.VMEM((_K,), jnp.int32),
            pltpu.VMEM((_K, _D), jnp.float32),
            pltpu.VMEM((_K, _D), jnp.float32),
            pltpu.VMEM((_K, _D), jnp.float32),
            pltpu.VMEM((_K, _D), jnp.float32),
            pltpu.VMEM_SHARED((_N, _D), jnp.float32),
            pltpu.SemaphoreType.DMA,
            pltpu.SemaphoreType.DMA,
            pltpu.SemaphoreType.DMA,
            pltpu.SemaphoreType.DMA,
        ],
    )
    partial = sc_scatter(wp, x, esrc, edst, zeros)

    out = pl.pallas_call(
        _post_body,
        grid=(_N // _NODE_BLK,),
        in_specs=[
            pl.BlockSpec((_NC, _NODE_BLK, _D), lambda i: (0, i, 0)),
            pl.BlockSpec((_NODE_BLK, _D), lambda i: (i, 0)),
            pl.BlockSpec((_D, _D), lambda i: (0, 0)),
        ],
        out_specs=pl.BlockSpec((_NODE_BLK, _D), lambda i: (i, 0)),
        out_shape=jax.ShapeDtypeStruct((_N, _D), jnp.float32),
    )(partial, si, W_lin2)
    return out
